# Initial kernel scaffold; baseline (speedup 1.0000x reference)
import jax
import jax.numpy as jnp
from jax.experimental import pallas as pl


def kernel(fea, params):
    raise NotImplementedError("write your pallas kernel here")


# speedup vs baseline: 9.6972x; 9.6972x over previous
import functools

import jax
import jax.numpy as jnp
from jax import lax
from jax.experimental import pallas as pl
from jax.experimental.pallas import tpu as pltpu
from jax.experimental.pallas import tpu_sc as plsc

KN = 16
NP = 4096
BB = 2
BN = BB * NP
BNK = BN * KN

_ENC_SPECS = [(3, 16, 24), (24, 16, 32), (32, 16, 48), (48, 24, 48),
              (48, 24, 64), (64, 24, 64), (64, 24, 128), (128, 32, 128)]
_DEC_SPECS = [(96, 128, 128), (128, 128, 128)]


def _c16(x):
    return (x + 15) // 16 * 16


_PB = 256


def _knn_body(xyz_ref, xyzt_ref, out_ref, dist_ref):
    b = pl.program_id(0)
    x = xyz_ref[0]
    xt = xyzt_ref[0]
    d2r = jnp.sum(x * x, axis=1, keepdims=True)
    d2c = jnp.sum(xt * xt, axis=0, keepdims=True)
    mm = jnp.dot(x, xt, preferred_element_type=jnp.float32)
    dist_ref[...] = d2r + d2c - 2.0 * mm
    iota = lax.broadcasted_iota(jnp.int32, (_PB, NP), 1)
    cols = []
    for _ in range(KN):
        dist = dist_ref[...]
        m = jnp.min(dist, axis=1, keepdims=True)
        sel = jnp.where(dist == m, iota, NP)
        j = jnp.min(sel, axis=1, keepdims=True)
        cols.append(j)
        dist_ref[...] = jnp.where(iota == j, jnp.inf, dist)
    out_ref[0] = jnp.concatenate(cols, axis=1) + b * NP


def _knn(xyz, xyzt):
    return pl.pallas_call(
        _knn_body,
        grid=(BB, NP // _PB),
        in_specs=[
            pl.BlockSpec((1, _PB, 3), lambda b, i: (b, i, 0)),
            pl.BlockSpec((1, 3, NP), lambda b, i: (b, 0, 0)),
        ],
        out_specs=pl.BlockSpec((1, _PB, KN), lambda b, i: (b, i, 0)),
        out_shape=jax.ShapeDtypeStruct((BB, NP, KN), jnp.int32),
        scratch_shapes=[pltpu.VMEM((_PB, NP), jnp.float32)],
    )(xyz, xyzt)


_NW = 32
_IPW = BNK // _NW
_GRP = 512
_RPS = 64
_SPG = _GRP // _RPS
_NGRP = _IPW // _GRP


def _sc_gather(table, idxf):
    mesh = plsc.VectorSubcoreMesh(core_axis_name="c", subcore_axis_name="s")

    @functools.partial(
        pl.kernel,
        mesh=mesh,
        out_type=jax.ShapeDtypeStruct((BNK, 128), jnp.float32),
        scratch_types=[
            pltpu.VMEM((_IPW,), jnp.int32),
            pltpu.VMEM((_GRP, 128), jnp.float32),
            pltpu.SemaphoreType.DMA,
        ],
    )
    def gk(table_hbm, idx_hbm, out_hbm, idx_v, rows_v, sem):
        wid = lax.axis_index("s") * 2 + lax.axis_index("c")
        base = wid * _IPW
        pltpu.sync_copy(idx_hbm.at[pl.ds(base, _IPW)], idx_v)

        def group(g, carry):
            gb = g * _GRP
            copies = [
                pltpu.async_copy(
                    table_hbm.at[idx_v.at[pl.ds(gb + j * _RPS, _RPS)]],
                    rows_v.at[pl.ds(j * _RPS, _RPS)],
                    sem,
                )
                for j in range(_SPG)
            ]
            for c in copies:
                c.wait()
            pltpu.sync_copy(rows_v, out_hbm.at[pl.ds(base + gb, _GRP)])
            return carry

        lax.fori_loop(0, _NGRP, group, 0)

    return gk(table, idxf)


_PR = 512


def _raw_body(nb_ref, xyz_ref, out_ref):
    nb = nb_ref[...][:, :16]
    ctr = jnp.broadcast_to(
        xyz_ref[...][:, :16].reshape(_PR, 1, 16), (_PR, KN, 16)
    ).reshape(_PR * KN, 16)
    rel = nb - ctr
    d = jnp.sqrt(jnp.sum(rel * rel, axis=1, keepdims=True))
    z = jnp.zeros((_PR * KN, 6), jnp.float32)
    out_ref[...] = jnp.concatenate(
        [ctr[:, :3], nb[:, :3], rel[:, :3], d, z], axis=1)


def _raw(nbxyz, xyzp):
    return pl.pallas_call(
        _raw_body,
        grid=(BN // _PR,),
        in_specs=[
            pl.BlockSpec((_PR * KN, 128), lambda i: (i, 0)),
            pl.BlockSpec((_PR, 128), lambda i: (i, 0)),
        ],
        out_specs=pl.BlockSpec((_PR * KN, 16), lambda i: (i, 0)),
        out_shape=jax.ShapeDtypeStruct((BNK, 16), jnp.float32),
    )(nbxyz, xyzp)


_PL = 256


def _lfa_body(nb_ref, raw_ref, fea_ref, wrel_ref, brel_ref, watt_ref,
              wout_ref, bout_ref, wsc_ref, bsc_ref, out_ref, cat_ref, *,
              in_c, nb_c, out_c):
    c = in_c + nb_c
    rel_enc = jnp.dot(raw_ref[...], wrel_ref[...],
                      preferred_element_type=jnp.float32) + brel_ref[...]
    rel_enc = jnp.where(rel_enc >= 0, rel_enc, 0.2 * rel_enc)
    nb = nb_ref[...][:, :in_c]
    cat_ref[:, :in_c] = nb
    cat_ref[:, in_c:] = rel_enc
    cat = cat_ref[...]
    logits = jnp.dot(cat, watt_ref[...],
                     preferred_element_type=jnp.float32)
    l3 = logits.reshape(_PL, KN, c)
    m = jnp.max(l3, axis=1, keepdims=True)
    e = jnp.exp(l3 - m)
    s = e[:, 0, :]
    for k in range(1, KN):
        s = s + e[:, k, :]
    att = e / s[:, None, :]
    c3 = cat.reshape(_PL, KN, c)
    prod = att * c3
    agg = prod[:, 0, :]
    for k in range(1, KN):
        agg = agg + prod[:, k, :]
    out = (jnp.dot(agg, wout_ref[...], preferred_element_type=jnp.float32)
           + bout_ref[...]
           + jnp.dot(fea_ref[...][:, :in_c], wsc_ref[...],
                     preferred_element_type=jnp.float32)
           + bsc_ref[...])
    out = jnp.where(out >= 0, out, 0.2 * out)
    if out_c < 128:
        out = jnp.concatenate(
            [out, jnp.zeros((_PL, 128 - out_c), jnp.float32)], axis=1)
    out_ref[...] = out


def _lfa_layer(fea, nbg, raw, p, in_c, nb_c, out_c):
    c = in_c + nb_c
    wrel = jnp.concatenate(
        [p["W_rel"], jnp.zeros((6, nb_c), jnp.float32)], axis=0)

    full = lambda i: (0, 0)
    return pl.pallas_call(
        functools.partial(_lfa_body, in_c=in_c, nb_c=nb_c, out_c=out_c),
        grid=(BN // _PL,),
        in_specs=[
            pl.BlockSpec((_PL * KN, 128), lambda i: (i, 0)),
            pl.BlockSpec((_PL * KN, 16), lambda i: (i, 0)),
            pl.BlockSpec((_PL, 128), lambda i: (i, 0)),
            pl.BlockSpec((16, nb_c), full),
            pl.BlockSpec((1, nb_c), full),
            pl.BlockSpec((c, c), full),
            pl.BlockSpec((c, out_c), full),
            pl.BlockSpec((1, out_c), full),
            pl.BlockSpec((in_c, out_c), full),
            pl.BlockSpec((1, out_c), full),
        ],
        out_specs=pl.BlockSpec((_PL, 128), lambda i: (i, 0)),
        out_shape=jax.ShapeDtypeStruct((BN, 128), jnp.float32),
        scratch_shapes=[pltpu.VMEM((_PL * KN, c), jnp.float32)],
    )(nbg, raw, fea, wrel, p["b_rel"][None, :], p["W_att"], p["W_out"],
      p["b_out"][None, :], p["W_sc"], p["b_sc"][None, :])


_PH = 512


def _enc_head_body(x_ref, w1_ref, b1_ref, w2_ref, b2_ref, out_ref):
    h = jnp.dot(x_ref[...][:, :128], w1_ref[...],
                preferred_element_type=jnp.float32) + b1_ref[...]
    h = jnp.where(h >= 0, h, 0.2 * h)
    h = jnp.dot(h, w2_ref[...], preferred_element_type=jnp.float32) + b2_ref[...]
    r = jnp.round(jax.nn.sigmoid(h))
    out_ref[...] = jnp.concatenate(
        [r, jnp.zeros((_PH, 32), jnp.float32)], axis=1)


def _dec_head_body(x_ref, w1_ref, b1_ref, w2_ref, b2_ref, out_ref):
    h = jnp.dot(x_ref[...], w1_ref[...],
                preferred_element_type=jnp.float32) + b1_ref[...]
    h = jnp.where(h >= 0, h, 0.2 * h)
    out_ref[...] = jnp.dot(h, w2_ref[...],
                           preferred_element_type=jnp.float32) + b2_ref[...]


def _head(body, x, hp, oc, ow):
    full = lambda i: (0, 0)
    return pl.pallas_call(
        body,
        grid=(BN // _PH,),
        in_specs=[
            pl.BlockSpec((_PH, 128), lambda i: (i, 0)),
            pl.BlockSpec((128, 128), full),
            pl.BlockSpec((1, 128), full),
            pl.BlockSpec((128, oc), full),
            pl.BlockSpec((1, oc), full),
        ],
        out_specs=pl.BlockSpec((_PH, ow), lambda i: (i, 0)),
        out_shape=jax.ShapeDtypeStruct((BN, ow), jnp.float32),
    )(x, hp["W1"], hp["b1"][None, :], hp["W2"], hp["b2"][None, :])


def kernel(fea, params):
    xyz = fea[..., :3]
    xyzt = jnp.transpose(xyz, (0, 2, 1))
    idxf = _knn(xyz, xyzt).reshape(BNK)

    xyzp = jnp.concatenate(
        [xyz.reshape(BN, 3), jnp.zeros((BN, 125), jnp.float32)], axis=1)
    nbxyz = _sc_gather(xyzp, idxf)
    raw = _raw(nbxyz, xyzp)

    h = xyzp
    for li, p in enumerate(params["enc"]):
        in_c, nb_c, out_c = _ENC_SPECS[li]
        nbg = _sc_gather(h, idxf)
        h = _lfa_layer(h, nbg, raw, p, in_c, nb_c, out_c)

    hq = _head(_enc_head_body, h, params["enc_out"], 96, 128)
    round_fea = hq[:, :96]

    h = hq
    for li, p in enumerate(params["dec"]):
        in_c, nb_c, out_c = _DEC_SPECS[li]
        nbg = _sc_gather(h, idxf)
        h = _lfa_layer(h, nbg, raw, p, in_c, nb_c, out_c)

    dec = _head(_dec_head_body, h, params["dec_out"], 3, 3)

    return (round_fea.reshape(BB, NP, 96), dec.reshape(BB, NP, 3))
